# baseline (speedup 1.0000x reference)
import functools

import numpy as np
import jax
import jax.numpy as jnp
from jax import lax
from jax.experimental import pallas as pl
from jax.experimental.pallas import tpu as pltpu
from jax.experimental.pallas import tpu_sc as plsc

NUM_CLASSES = 200
ACT_LEN = NUM_CLASSES + 1
COMP_LEN = NUM_CLASSES
REG_LEN = NUM_CLASSES * 2
NUM_MULT = 5
FEAT_DIM = ACT_LEN + NUM_MULT * (COMP_LEN + REG_LEN)
T_TOTAL = 8192
NUM_TICKS = 128

F_PAD = 3328
N_CHUNKS = F_PAD // 128
BT = 256
T_STEPS = T_TOTAL // BT
P_ROWS = (T_STEPS + 1) * BT

NC, NS, L = 2, 16, 16
NW = NC * NS
PROPS_PER_W = NUM_TICKS // NW

ACT_PAD, COMP_PAD, REG_PAD = 208, 208, 416

U_L0, U_R0, U_L1, U_M1, U_R1, U_L2, U_R2 = range(7)

_TERMS = (
    (U_L0, U_R0, 1, 201, 1201),
    (U_L1, U_R1, 2, 401, 1601),
    (U_L1, U_M1, 3, 601, 2001),
    (U_M1, U_R1, 4, 801, 2401),
    (U_L2, U_R2, 5, 1001, 2801),
)
N_COEF = 6


def _build_segments():
    segs = []
    seg_of = {}
    def add(u, c0, c1, key):
        seg_of[key] = len(segs)
        segs.append((u, c0, c1 - c0 + 1))
    add(U_L1, 0, (ACT_LEN - 1) // 128, ("act", 0, "lo"))
    add(U_R1, 0, (ACT_LEN - 1) // 128, ("act", 0, "hi"))
    for j, (lo_u, hi_u, _ci, comp_b, reg_b) in enumerate(_TERMS):
        c0, c1 = comp_b // 128, (comp_b + COMP_LEN - 1) // 128
        add(lo_u, c0, c1, ("comp", j, "lo"))
        add(hi_u, c0, c1, ("comp", j, "hi"))
    for j, (lo_u, hi_u, _ci, comp_b, reg_b) in enumerate(_TERMS):
        c0, c1 = reg_b // 128, (reg_b + REG_LEN - 1) // 128
        add(lo_u, c0, c1, ("reg", j, "lo"))
        add(hi_u, c0, c1, ("reg", j, "hi"))
    bases, acc = [], 0
    for (_u, _c0, n) in segs:
        bases.append(acc)
        acc += n
    return segs, seg_of, bases, acc


_SEGS, _SEG_OF, _SEG_BASE, N_GATHER = _build_segments()


BF = 256
F_BLOCKS = F_PAD // BF
ZERO_ROW = T_STEPS * BT


def _prefix_body(x_ref, p_ref, carry_ref):
    t = pl.program_id(1)

    @pl.when(t == 0)
    def _():
        carry_ref[...] = jnp.zeros_like(carry_ref)

    @pl.when(t < T_STEPS)
    def _():
        xb = x_ref[...]
        row = lax.broadcasted_iota(jnp.int32, (BT, BT), 0)
        col = lax.broadcasted_iota(jnp.int32, (BT, BT), 1)
        tri = (col <= row).astype(jnp.float32)
        res = lax.dot_general(
            tri, xb, (((1,), (1,)), ((), ())),
            preferred_element_type=jnp.float32) + carry_ref[...]
        p_ref[...] = jnp.swapaxes(res.reshape(BT, BF // 128, 128), 0, 1)
        carry_ref[...] = res[BT - 1:BT, :]

    @pl.when(t == T_STEPS)
    def _():
        p_ref[...] = jnp.zeros_like(p_ref)


_prefix_call = pl.pallas_call(
    _prefix_body,
    grid=(F_BLOCKS, T_STEPS + 1),
    in_specs=[pl.BlockSpec(
        (BF, BT), lambda f, t: (f, jnp.minimum(t, T_STEPS - 1)))],
    out_specs=pl.BlockSpec((BF // 128, BT, 128), lambda f, t: (f, t, 0)),
    out_shape=jax.ShapeDtypeStruct((N_CHUNKS, P_ROWS, 128), jnp.float32),
    scratch_shapes=[pltpu.VMEM((1, BF), jnp.float32)],
    compiler_params=pltpu.CompilerParams(
        dimension_semantics=("arbitrary", "arbitrary")),
)


def _load_win(rows_v, seg_idx, rel_off):
    s = _SEG_BASE[seg_idx] * 128 + rel_off
    r0, c0 = divmod(s, 128)
    if c0 + L <= 128:
        return rows_v[r0, pl.ds(c0, L)]
    k = 128 - c0
    v0 = rows_v[r0, pl.ds(128 - L, L)]
    v1 = rows_v[r0 + 1, pl.ds(0, L)]
    lanes = lax.iota(jnp.int32, L)
    i0 = jnp.minimum(lanes + (c0 - (128 - L)), L - 1)
    i1 = jnp.maximum(lanes - k, 0)
    return jnp.where(lanes < k, _take16(v0, i0), _take16(v1, i1))


def _take16(v, idx):
    return lax.gather(
        v, idx[:, None],
        lax.GatherDimensionNumbers(
            offset_dims=(), collapsed_slice_dims=(0,), start_index_map=(0,)),
        slice_sizes=(1,),
        mode=lax.GatherScatterMode.PROMISE_IN_BOUNDS)


def _combine_body(p_hbm, idx_hbm, coefb_hbm, act_hbm, comp_hbm, reg_hbm,
                  idx_v, coefb_v, rows_v, act_v, comp_v, reg_v, sem):
    wid = lax.axis_index("s") * NC + lax.axis_index("c")
    pltpu.sync_copy(coefb_hbm.at[wid], coefb_v)

    for slot in range(PROPS_PER_W):
        pltpu.sync_copy(idx_hbm.at[wid, slot], idx_v)
        pltpu.async_copy(p_hbm.at[idx_v], rows_v, sem).wait()
        cbase = slot * N_COEF
        c_act = coefb_v[cbase + 0, :]
        s_lo, s_hi = _SEG_OF[("act", 0, "lo")], _SEG_OF[("act", 0, "hi")]
        for c in range(ACT_PAD // L):
            off = c * L
            hi = _load_win(rows_v, s_hi, off)
            lo = _load_win(rows_v, s_lo, off)
            act_v[slot, pl.ds(off, L)] = (hi - lo) * c_act
        for c in range(COMP_PAD // L):
            off = c * L
            acc = jnp.zeros((L,), jnp.float32)
            for j, (_lo_u, _hi_u, ci, comp_b, _reg_b) in enumerate(_TERMS):
                cf = coefb_v[cbase + ci, :]
                rel = comp_b - 128 * _SEGS[_SEG_OF[("comp", j, "lo")]][1] + off
                hi = _load_win(rows_v, _SEG_OF[("comp", j, "hi")], rel)
                lo = _load_win(rows_v, _SEG_OF[("comp", j, "lo")], rel)
                acc = acc + (hi - lo) * cf
            comp_v[slot, pl.ds(off, L)] = acc
        for c in range(REG_PAD // L):
            off = c * L
            acc = jnp.zeros((L,), jnp.float32)
            for j, (_lo_u, _hi_u, ci, _comp_b, reg_b) in enumerate(_TERMS):
                cf = coefb_v[cbase + ci, :]
                rel = reg_b - 128 * _SEGS[_SEG_OF[("reg", j, "lo")]][1] + off
                hi = _load_win(rows_v, _SEG_OF[("reg", j, "hi")], rel)
                lo = _load_win(rows_v, _SEG_OF[("reg", j, "lo")], rel)
                acc = acc + (hi - lo) * cf
            reg_v[slot, pl.ds(off, L)] = acc

    base = wid * PROPS_PER_W
    pltpu.sync_copy(act_v, act_hbm.at[pl.ds(base, PROPS_PER_W)])
    pltpu.sync_copy(comp_v, comp_hbm.at[pl.ds(base, PROPS_PER_W)])
    pltpu.sync_copy(reg_v, reg_hbm.at[pl.ds(base, PROPS_PER_W)])


@functools.cache
def _combine_call():
    return functools.partial(
        pl.kernel,
        mesh=plsc.VectorSubcoreMesh(core_axis_name="c", subcore_axis_name="s"),
        out_type=(
            jax.ShapeDtypeStruct((NUM_TICKS, ACT_PAD), jnp.float32),
            jax.ShapeDtypeStruct((NUM_TICKS, COMP_PAD), jnp.float32),
            jax.ShapeDtypeStruct((NUM_TICKS, REG_PAD), jnp.float32),
        ),
        scratch_types=[
            pltpu.VMEM((N_GATHER,), jnp.int32),
            pltpu.VMEM((PROPS_PER_W * N_COEF, L), jnp.float32),
            pltpu.VMEM((N_GATHER, 128), jnp.float32),
            pltpu.VMEM((PROPS_PER_W, ACT_PAD), jnp.float32),
            pltpu.VMEM((PROPS_PER_W, COMP_PAD), jnp.float32),
            pltpu.VMEM((PROPS_PER_W, REG_PAD), jnp.float32),
            pltpu.SemaphoreType.DMA,
        ],
    )(_combine_body)


def _boundaries(proposal_ticks, scale_factors):
    tk = proposal_ticks.astype(jnp.int32)
    t0, t1, t2, t3 = tk[:, 0], tk[:, 1], tk[:, 2], tk[:, 3]
    r0 = jnp.maximum(t0 + 1, t1)
    r1 = jnp.maximum(t1 + 1, t2)
    r2 = jnp.maximum(t2 + 1, t3)
    m1 = t1 + (r1 - t1) // 2
    rows = jnp.stack([t0, r0, t1, m1, r1, t2, r2], axis=1)

    f32 = jnp.float32
    inv = lambda a, b: 1.0 / jnp.maximum(b - a, 1).astype(f32)
    coefs = jnp.stack([
        inv(t1, r1),
        scale_factors[:, 0] * inv(t0, r0),
        inv(t1, r1),
        inv(t1, m1),
        inv(m1, r1),
        scale_factors[:, 1] * inv(t2, r2),
    ], axis=1)
    return rows, coefs


_GATHER_U = np.concatenate(
    [np.full(n, u, np.int32) for (u, c0, n) in _SEGS])
_GATHER_C = np.concatenate(
    [np.arange(c0, c0 + n, dtype=np.int32) for (u, c0, n) in _SEGS])


def kernel(x, proposal_ticks, scale_factors):
    p3 = _prefix_call(jnp.swapaxes(x, 0, 1))
    p_flat = p3.reshape(N_CHUNKS * P_ROWS, 128)

    rows, coefs = _boundaries(proposal_ticks, scale_factors)
    rows = jnp.where(rows >= 1, rows - 1, ZERO_ROW)
    gat = rows[:, _GATHER_U] + jnp.asarray(_GATHER_C * P_ROWS)[None, :]
    idx = gat.reshape(NW, PROPS_PER_W, N_GATHER)
    coefb = jnp.broadcast_to(
        coefs[:, :, None], (NUM_TICKS, N_COEF, L)
    ).reshape(NW, PROPS_PER_W * N_COEF, L)

    act, comp, reg = _combine_call()(p_flat, idx, coefb)
    return act[:, :ACT_LEN], comp[:, :COMP_LEN], reg[:, :REG_LEN]


# speedup vs baseline: 2.8181x; 2.8181x over previous
import functools

import numpy as np
import jax
import jax.numpy as jnp
from jax import lax
from jax.experimental import pallas as pl
from jax.experimental.pallas import tpu as pltpu
from jax.experimental.pallas import tpu_sc as plsc

NUM_CLASSES = 200
ACT_LEN = NUM_CLASSES + 1
COMP_LEN = NUM_CLASSES
REG_LEN = NUM_CLASSES * 2
NUM_MULT = 5
FEAT_DIM = ACT_LEN + NUM_MULT * (COMP_LEN + REG_LEN)
T_TOTAL = 8192
NUM_TICKS = 128

F_PAD = 3328
N_CHUNKS = F_PAD // 128
BT = 256
T_STEPS = T_TOTAL // BT
P_ROWS = (T_STEPS + 1) * BT

NC, NS, L = 2, 16, 16
NW = NC * NS
PROPS_PER_W = NUM_TICKS // NW

ACT_PAD, COMP_PAD, REG_PAD = 208, 208, 416

U_L0, U_R0, U_L1, U_M1, U_R1, U_L2, U_R2 = range(7)

_TERMS = (
    (U_L0, U_R0, 1, 201, 1201),
    (U_L1, U_R1, 2, 401, 1601),
    (U_L1, U_M1, 3, 601, 2001),
    (U_M1, U_R1, 4, 801, 2401),
    (U_L2, U_R2, 5, 1001, 2801),
)
N_COEF = 6


def _build_segments():
    segs = []
    seg_of = {}
    def add(u, c0, c1, key):
        seg_of[key] = len(segs)
        segs.append((u, c0, c1 - c0 + 1))
    add(U_L1, 0, (ACT_LEN - 1) // 128, ("act", 0, "lo"))
    add(U_R1, 0, (ACT_LEN - 1) // 128, ("act", 0, "hi"))
    for j, (lo_u, hi_u, _ci, comp_b, reg_b) in enumerate(_TERMS):
        c0, c1 = comp_b // 128, (comp_b + COMP_LEN - 1) // 128
        add(lo_u, c0, c1, ("comp", j, "lo"))
        add(hi_u, c0, c1, ("comp", j, "hi"))
    for j, (lo_u, hi_u, _ci, comp_b, reg_b) in enumerate(_TERMS):
        c0, c1 = reg_b // 128, (reg_b + REG_LEN - 1) // 128
        add(lo_u, c0, c1, ("reg", j, "lo"))
        add(hi_u, c0, c1, ("reg", j, "hi"))
    bases, acc = [], 0
    for (_u, _c0, n) in segs:
        bases.append(acc)
        acc += n
    return segs, seg_of, bases, acc


_SEGS, _SEG_OF, _SEG_BASE, N_GATHER = _build_segments()


ZERO_ROW = T_STEPS * BT


def _prefix_body(x_ref, p_ref, carry_ref):
    t = pl.program_id(0)

    @pl.when(t == 0)
    def _():
        carry_ref[...] = jnp.zeros_like(carry_ref)

    @pl.when(t < T_STEPS)
    def _():
        xb_t = jnp.swapaxes(x_ref[...], 0, 1)
        row = lax.broadcasted_iota(jnp.int32, (BT, BT), 0)
        col = lax.broadcasted_iota(jnp.int32, (BT, BT), 1)
        tri = (col <= row).astype(jnp.float32)
        res = jnp.dot(tri, xb_t,
                      preferred_element_type=jnp.float32) + carry_ref[...]
        p_ref[...] = jnp.swapaxes(res.reshape(BT, N_CHUNKS, 128), 0, 1)
        carry_ref[...] = res[BT - 1:BT, :]

    @pl.when(t == T_STEPS)
    def _():
        p_ref[...] = jnp.zeros_like(p_ref)


_prefix_call = pl.pallas_call(
    _prefix_body,
    grid=(T_STEPS + 1,),
    in_specs=[pl.BlockSpec(
        (F_PAD, BT), lambda t: (0, jnp.minimum(t, T_STEPS - 1)))],
    out_specs=pl.BlockSpec((N_CHUNKS, BT, 128), lambda t: (0, t, 0)),
    out_shape=jax.ShapeDtypeStruct((N_CHUNKS, P_ROWS, 128), jnp.float32),
    scratch_shapes=[pltpu.VMEM((1, F_PAD), jnp.float32)],
    compiler_params=pltpu.CompilerParams(
        dimension_semantics=("arbitrary",)),
)


def _load_win(rows_v, seg_idx, rel_off):
    s = _SEG_BASE[seg_idx] * 128 + rel_off
    r0, c0 = divmod(s, 128)
    if c0 + L <= 128:
        return rows_v[r0, pl.ds(c0, L)]
    k = 128 - c0
    v0 = rows_v[r0, pl.ds(128 - L, L)]
    v1 = rows_v[r0 + 1, pl.ds(0, L)]
    lanes = lax.iota(jnp.int32, L)
    i0 = jnp.minimum(lanes + (c0 - (128 - L)), L - 1)
    i1 = jnp.maximum(lanes - k, 0)
    return jnp.where(lanes < k, _take16(v0, i0), _take16(v1, i1))


def _take16(v, idx):
    return lax.gather(
        v, idx[:, None],
        lax.GatherDimensionNumbers(
            offset_dims=(), collapsed_slice_dims=(0,), start_index_map=(0,)),
        slice_sizes=(1,),
        mode=lax.GatherScatterMode.PROMISE_IN_BOUNDS)


def _combine_body(p_hbm, idx_hbm, coefb_hbm, act_hbm, comp_hbm, reg_hbm,
                  idx_v, coefb_v, rows_v, act_v, comp_v, reg_v, sem):
    wid = lax.axis_index("s") * NC + lax.axis_index("c")
    pltpu.sync_copy(coefb_hbm.at[wid], coefb_v)

    for slot in range(PROPS_PER_W):
        pltpu.sync_copy(idx_hbm.at[wid, slot], idx_v)
        pltpu.async_copy(p_hbm.at[idx_v], rows_v, sem).wait()
        cbase = slot * N_COEF
        c_act = coefb_v[cbase + 0, :]
        s_lo, s_hi = _SEG_OF[("act", 0, "lo")], _SEG_OF[("act", 0, "hi")]
        for c in range(ACT_PAD // L):
            off = c * L
            hi = _load_win(rows_v, s_hi, off)
            lo = _load_win(rows_v, s_lo, off)
            act_v[slot, pl.ds(off, L)] = (hi - lo) * c_act
        for c in range(COMP_PAD // L):
            off = c * L
            acc = jnp.zeros((L,), jnp.float32)
            for j, (_lo_u, _hi_u, ci, comp_b, _reg_b) in enumerate(_TERMS):
                cf = coefb_v[cbase + ci, :]
                rel = comp_b - 128 * _SEGS[_SEG_OF[("comp", j, "lo")]][1] + off
                hi = _load_win(rows_v, _SEG_OF[("comp", j, "hi")], rel)
                lo = _load_win(rows_v, _SEG_OF[("comp", j, "lo")], rel)
                acc = acc + (hi - lo) * cf
            comp_v[slot, pl.ds(off, L)] = acc
        for c in range(REG_PAD // L):
            off = c * L
            acc = jnp.zeros((L,), jnp.float32)
            for j, (_lo_u, _hi_u, ci, _comp_b, reg_b) in enumerate(_TERMS):
                cf = coefb_v[cbase + ci, :]
                rel = reg_b - 128 * _SEGS[_SEG_OF[("reg", j, "lo")]][1] + off
                hi = _load_win(rows_v, _SEG_OF[("reg", j, "hi")], rel)
                lo = _load_win(rows_v, _SEG_OF[("reg", j, "lo")], rel)
                acc = acc + (hi - lo) * cf
            reg_v[slot, pl.ds(off, L)] = acc

    base = wid * PROPS_PER_W
    pltpu.sync_copy(act_v, act_hbm.at[pl.ds(base, PROPS_PER_W)])
    pltpu.sync_copy(comp_v, comp_hbm.at[pl.ds(base, PROPS_PER_W)])
    pltpu.sync_copy(reg_v, reg_hbm.at[pl.ds(base, PROPS_PER_W)])


@functools.cache
def _combine_call():
    return functools.partial(
        pl.kernel,
        mesh=plsc.VectorSubcoreMesh(core_axis_name="c", subcore_axis_name="s"),
        out_type=(
            jax.ShapeDtypeStruct((NUM_TICKS, ACT_PAD), jnp.float32),
            jax.ShapeDtypeStruct((NUM_TICKS, COMP_PAD), jnp.float32),
            jax.ShapeDtypeStruct((NUM_TICKS, REG_PAD), jnp.float32),
        ),
        scratch_types=[
            pltpu.VMEM((N_GATHER,), jnp.int32),
            pltpu.VMEM((PROPS_PER_W * N_COEF, L), jnp.float32),
            pltpu.VMEM((N_GATHER, 128), jnp.float32),
            pltpu.VMEM((PROPS_PER_W, ACT_PAD), jnp.float32),
            pltpu.VMEM((PROPS_PER_W, COMP_PAD), jnp.float32),
            pltpu.VMEM((PROPS_PER_W, REG_PAD), jnp.float32),
            pltpu.SemaphoreType.DMA,
        ],
    )(_combine_body)


def _boundaries(proposal_ticks, scale_factors):
    tk = proposal_ticks.astype(jnp.int32)
    t0, t1, t2, t3 = tk[:, 0], tk[:, 1], tk[:, 2], tk[:, 3]
    r0 = jnp.maximum(t0 + 1, t1)
    r1 = jnp.maximum(t1 + 1, t2)
    r2 = jnp.maximum(t2 + 1, t3)
    m1 = t1 + (r1 - t1) // 2
    rows = jnp.stack([t0, r0, t1, m1, r1, t2, r2], axis=1)

    f32 = jnp.float32
    inv = lambda a, b: 1.0 / jnp.maximum(b - a, 1).astype(f32)
    coefs = jnp.stack([
        inv(t1, r1),
        scale_factors[:, 0] * inv(t0, r0),
        inv(t1, r1),
        inv(t1, m1),
        inv(m1, r1),
        scale_factors[:, 1] * inv(t2, r2),
    ], axis=1)
    return rows, coefs


_GATHER_U = np.concatenate(
    [np.full(n, u, np.int32) for (u, c0, n) in _SEGS])
_GATHER_C = np.concatenate(
    [np.arange(c0, c0 + n, dtype=np.int32) for (u, c0, n) in _SEGS])


def kernel(x, proposal_ticks, scale_factors):
    p3 = _prefix_call(jnp.swapaxes(x, 0, 1))
    p_flat = p3.reshape(N_CHUNKS * P_ROWS, 128)

    rows, coefs = _boundaries(proposal_ticks, scale_factors)
    rows = jnp.where(rows >= 1, rows - 1, ZERO_ROW)
    gat = rows[:, _GATHER_U] + jnp.asarray(_GATHER_C * P_ROWS)[None, :]
    idx = gat.reshape(NW, PROPS_PER_W, N_GATHER)
    coefb = jnp.broadcast_to(
        coefs[:, :, None], (NUM_TICKS, N_COEF, L)
    ).reshape(NW, PROPS_PER_W * N_COEF, L)

    act, comp, reg = _combine_call()(p_flat, idx, coefb)
    return act[:, :ACT_LEN], comp[:, :COMP_LEN], reg[:, :REG_LEN]


# speedup vs baseline: 2.8822x; 1.0227x over previous
import functools

import numpy as np
import jax
import jax.numpy as jnp
from jax import lax
from jax.experimental import pallas as pl
from jax.experimental.pallas import tpu as pltpu
from jax.experimental.pallas import tpu_sc as plsc

NUM_CLASSES = 200
ACT_LEN = NUM_CLASSES + 1
COMP_LEN = NUM_CLASSES
REG_LEN = NUM_CLASSES * 2
NUM_MULT = 5
FEAT_DIM = ACT_LEN + NUM_MULT * (COMP_LEN + REG_LEN)
T_TOTAL = 8192
NUM_TICKS = 128

F_PAD = 3328
N_CHUNKS = F_PAD // 128
BT = 256
T_STEPS = T_TOTAL // BT
P_ROWS = (T_STEPS + 1) * BT

NC, NS, L = 2, 16, 16
NW = NC * NS
PROPS_PER_W = NUM_TICKS // NW

ACT_PAD, COMP_PAD, REG_PAD = 208, 208, 416

U_L0, U_R0, U_L1, U_M1, U_R1, U_L2, U_R2 = range(7)

_TERMS = (
    (U_L0, U_R0, 1, 201, 1201),
    (U_L1, U_R1, 2, 401, 1601),
    (U_L1, U_M1, 3, 601, 2001),
    (U_M1, U_R1, 4, 801, 2401),
    (U_L2, U_R2, 5, 1001, 2801),
)
N_COEF = 6


def _build_segments():
    segs = []
    seg_of = {}
    def add(u, c0, c1, key):
        seg_of[key] = len(segs)
        segs.append((u, c0, c1 - c0 + 1))
    add(U_L1, 0, (ACT_LEN - 1) // 128, ("act", 0, "lo"))
    add(U_R1, 0, (ACT_LEN - 1) // 128, ("act", 0, "hi"))
    for j, (lo_u, hi_u, _ci, comp_b, reg_b) in enumerate(_TERMS):
        c0, c1 = comp_b // 128, (comp_b + COMP_LEN - 1) // 128
        add(lo_u, c0, c1, ("comp", j, "lo"))
        add(hi_u, c0, c1, ("comp", j, "hi"))
    for j, (lo_u, hi_u, _ci, comp_b, reg_b) in enumerate(_TERMS):
        c0, c1 = reg_b // 128, (reg_b + REG_LEN - 1) // 128
        add(lo_u, c0, c1, ("reg", j, "lo"))
        add(hi_u, c0, c1, ("reg", j, "hi"))
    bases, acc = [], 0
    for (_u, _c0, n) in segs:
        bases.append(acc)
        acc += n
    return segs, seg_of, bases, acc


_SEGS, _SEG_OF, _SEG_BASE, N_GATHER = _build_segments()


ZERO_ROW = T_STEPS * BT


def _prefix_body(x_ref, p_ref, carry_ref):
    t = pl.program_id(0)

    @pl.when(t == 0)
    def _():
        carry_ref[...] = jnp.zeros_like(carry_ref)

    @pl.when(t < T_STEPS)
    def _():
        xb_t = jnp.swapaxes(x_ref[...], 0, 1)
        row = lax.broadcasted_iota(jnp.int32, (BT, BT), 0)
        col = lax.broadcasted_iota(jnp.int32, (BT, BT), 1)
        tri = (col <= row).astype(jnp.float32)
        res = jnp.dot(tri, xb_t,
                      preferred_element_type=jnp.float32) + carry_ref[...]
        p_ref[...] = jnp.swapaxes(res.reshape(BT, N_CHUNKS, 128), 0, 1)
        carry_ref[...] = res[BT - 1:BT, :]

    @pl.when(t == T_STEPS)
    def _():
        p_ref[...] = jnp.zeros_like(p_ref)


_prefix_call = pl.pallas_call(
    _prefix_body,
    grid=(T_STEPS + 1,),
    in_specs=[pl.BlockSpec(
        (F_PAD, BT), lambda t: (0, jnp.minimum(t, T_STEPS - 1)))],
    out_specs=pl.BlockSpec((N_CHUNKS, BT, 128), lambda t: (0, t, 0)),
    out_shape=jax.ShapeDtypeStruct((N_CHUNKS, P_ROWS, 128), jnp.float32),
    scratch_shapes=[pltpu.VMEM((1, F_PAD), jnp.float32)],
    compiler_params=pltpu.CompilerParams(
        dimension_semantics=("arbitrary",)),
)


def _load_win(rows_v, seg_idx, rel_off):
    s = _SEG_BASE[seg_idx] * 128 + rel_off
    r0, c0 = divmod(s, 128)
    if c0 + L <= 128:
        return rows_v[r0, pl.ds(c0, L)]
    k = 128 - c0
    v0 = rows_v[r0, pl.ds(128 - L, L)]
    v1 = rows_v[r0 + 1, pl.ds(0, L)]
    lanes = lax.iota(jnp.int32, L)
    i0 = jnp.minimum(lanes + (c0 - (128 - L)), L - 1)
    i1 = jnp.maximum(lanes - k, 0)
    return jnp.where(lanes < k, _take16(v0, i0), _take16(v1, i1))


def _take16(v, idx):
    return lax.gather(
        v, idx[:, None],
        lax.GatherDimensionNumbers(
            offset_dims=(), collapsed_slice_dims=(0,), start_index_map=(0,)),
        slice_sizes=(1,),
        mode=lax.GatherScatterMode.PROMISE_IN_BOUNDS)


def _combine_body(p_hbm, idx_hbm, coefb_hbm, act_hbm, comp_hbm, reg_hbm,
                  idx_v0, idx_v1, coefb_v, rows_v0, rows_v1,
                  act_v, comp_v, reg_v, sem0, sem1):
    wid = lax.axis_index("s") * NC + lax.axis_index("c")
    pltpu.sync_copy(coefb_hbm.at[wid], coefb_v)

    idx_bufs = (idx_v0, idx_v1)
    row_bufs = (rows_v0, rows_v1)
    sems = (sem0, sem1)
    pltpu.sync_copy(idx_hbm.at[wid, 0], idx_v0)
    copies = [pltpu.async_copy(p_hbm.at[idx_v0], rows_v0, sem0)]
    for slot in range(PROPS_PER_W):
        if slot + 1 < PROPS_PER_W:
            nb = (slot + 1) % 2
            pltpu.sync_copy(idx_hbm.at[wid, slot + 1], idx_bufs[nb])
            copies.append(pltpu.async_copy(
                p_hbm.at[idx_bufs[nb]], row_bufs[nb], sems[nb]))
        copies[slot].wait()
        rows_v = row_bufs[slot % 2]
        cbase = slot * N_COEF
        c_act = coefb_v[cbase + 0, :]
        s_lo, s_hi = _SEG_OF[("act", 0, "lo")], _SEG_OF[("act", 0, "hi")]
        for c in range(ACT_PAD // L):
            off = c * L
            hi = _load_win(rows_v, s_hi, off)
            lo = _load_win(rows_v, s_lo, off)
            act_v[slot, pl.ds(off, L)] = (hi - lo) * c_act
        for c in range(COMP_PAD // L):
            off = c * L
            acc = jnp.zeros((L,), jnp.float32)
            for j, (_lo_u, _hi_u, ci, comp_b, _reg_b) in enumerate(_TERMS):
                cf = coefb_v[cbase + ci, :]
                rel = comp_b - 128 * _SEGS[_SEG_OF[("comp", j, "lo")]][1] + off
                hi = _load_win(rows_v, _SEG_OF[("comp", j, "hi")], rel)
                lo = _load_win(rows_v, _SEG_OF[("comp", j, "lo")], rel)
                acc = acc + (hi - lo) * cf
            comp_v[slot, pl.ds(off, L)] = acc
        for c in range(REG_PAD // L):
            off = c * L
            acc = jnp.zeros((L,), jnp.float32)
            for j, (_lo_u, _hi_u, ci, _comp_b, reg_b) in enumerate(_TERMS):
                cf = coefb_v[cbase + ci, :]
                rel = reg_b - 128 * _SEGS[_SEG_OF[("reg", j, "lo")]][1] + off
                hi = _load_win(rows_v, _SEG_OF[("reg", j, "hi")], rel)
                lo = _load_win(rows_v, _SEG_OF[("reg", j, "lo")], rel)
                acc = acc + (hi - lo) * cf
            reg_v[slot, pl.ds(off, L)] = acc

    base = wid * PROPS_PER_W
    pltpu.sync_copy(act_v, act_hbm.at[pl.ds(base, PROPS_PER_W)])
    pltpu.sync_copy(comp_v, comp_hbm.at[pl.ds(base, PROPS_PER_W)])
    pltpu.sync_copy(reg_v, reg_hbm.at[pl.ds(base, PROPS_PER_W)])


@functools.cache
def _combine_call():
    return functools.partial(
        pl.kernel,
        mesh=plsc.VectorSubcoreMesh(core_axis_name="c", subcore_axis_name="s"),
        out_type=(
            jax.ShapeDtypeStruct((NUM_TICKS, ACT_PAD), jnp.float32),
            jax.ShapeDtypeStruct((NUM_TICKS, COMP_PAD), jnp.float32),
            jax.ShapeDtypeStruct((NUM_TICKS, REG_PAD), jnp.float32),
        ),
        scratch_types=[
            pltpu.VMEM((N_GATHER,), jnp.int32),
            pltpu.VMEM((N_GATHER,), jnp.int32),
            pltpu.VMEM((PROPS_PER_W * N_COEF, L), jnp.float32),
            pltpu.VMEM((N_GATHER, 128), jnp.float32),
            pltpu.VMEM((N_GATHER, 128), jnp.float32),
            pltpu.VMEM((PROPS_PER_W, ACT_PAD), jnp.float32),
            pltpu.VMEM((PROPS_PER_W, COMP_PAD), jnp.float32),
            pltpu.VMEM((PROPS_PER_W, REG_PAD), jnp.float32),
            pltpu.SemaphoreType.DMA,
            pltpu.SemaphoreType.DMA,
        ],
    )(_combine_body)


def _boundaries(proposal_ticks, scale_factors):
    tk = proposal_ticks.astype(jnp.int32)
    t0, t1, t2, t3 = tk[:, 0], tk[:, 1], tk[:, 2], tk[:, 3]
    r0 = jnp.maximum(t0 + 1, t1)
    r1 = jnp.maximum(t1 + 1, t2)
    r2 = jnp.maximum(t2 + 1, t3)
    m1 = t1 + (r1 - t1) // 2
    rows = jnp.stack([t0, r0, t1, m1, r1, t2, r2], axis=1)

    f32 = jnp.float32
    inv = lambda a, b: 1.0 / jnp.maximum(b - a, 1).astype(f32)
    coefs = jnp.stack([
        inv(t1, r1),
        scale_factors[:, 0] * inv(t0, r0),
        inv(t1, r1),
        inv(t1, m1),
        inv(m1, r1),
        scale_factors[:, 1] * inv(t2, r2),
    ], axis=1)
    return rows, coefs


_GATHER_U = np.concatenate(
    [np.full(n, u, np.int32) for (u, c0, n) in _SEGS])
_GATHER_C = np.concatenate(
    [np.arange(c0, c0 + n, dtype=np.int32) for (u, c0, n) in _SEGS])


def kernel(x, proposal_ticks, scale_factors):
    p3 = _prefix_call(jnp.swapaxes(x, 0, 1))
    p_flat = p3.reshape(N_CHUNKS * P_ROWS, 128)

    rows, coefs = _boundaries(proposal_ticks, scale_factors)
    rows = jnp.where(rows >= 1, rows - 1, ZERO_ROW)
    gat = rows[:, _GATHER_U] + jnp.asarray(_GATHER_C * P_ROWS)[None, :]
    idx = gat.reshape(NW, PROPS_PER_W, N_GATHER)
    coefb = jnp.broadcast_to(
        coefs[:, :, None], (NUM_TICKS, N_COEF, L)
    ).reshape(NW, PROPS_PER_W * N_COEF, L)

    act, comp, reg = _combine_call()(p_flat, idx, coefb)
    return act[:, :ACT_LEN], comp[:, :COMP_LEN], reg[:, :REG_LEN]


# speedup vs baseline: 2.9955x; 1.0393x over previous
import functools

import numpy as np
import jax
import jax.numpy as jnp
from jax import lax
from jax.experimental import pallas as pl
from jax.experimental.pallas import tpu as pltpu
from jax.experimental.pallas import tpu_sc as plsc

NUM_CLASSES = 200
ACT_LEN = NUM_CLASSES + 1
COMP_LEN = NUM_CLASSES
REG_LEN = NUM_CLASSES * 2
NUM_MULT = 5
FEAT_DIM = ACT_LEN + NUM_MULT * (COMP_LEN + REG_LEN)
T_TOTAL = 8192
NUM_TICKS = 128

F_PAD = 3328
N_CHUNKS = F_PAD // 128
BT = 256
T_STEPS = T_TOTAL // BT
P_ROWS = (T_STEPS + 1) * BT

NC, NS, L = 2, 16, 16
NW = NC * NS
PROPS_PER_W = NUM_TICKS // NW

ACT_PAD, COMP_PAD, REG_PAD = 208, 208, 416

U_L0, U_R0, U_L1, U_M1, U_R1, U_L2, U_R2 = range(7)

_TERMS = (
    (U_L0, U_R0, 1, 201, 1201),
    (U_L1, U_R1, 2, 401, 1601),
    (U_L1, U_M1, 3, 601, 2001),
    (U_M1, U_R1, 4, 801, 2401),
    (U_L2, U_R2, 5, 1001, 2801),
)
N_COEF = 6


def _build_segments():
    segs = []
    seg_of = {}
    def add(u, c0, c1, key):
        seg_of[key] = len(segs)
        segs.append((u, c0, c1 - c0 + 1))
    add(U_L1, 0, (ACT_LEN - 1) // 128, ("act", 0, "lo"))
    add(U_R1, 0, (ACT_LEN - 1) // 128, ("act", 0, "hi"))
    for j, (lo_u, hi_u, _ci, comp_b, reg_b) in enumerate(_TERMS):
        c0, c1 = comp_b // 128, (comp_b + COMP_LEN - 1) // 128
        add(lo_u, c0, c1, ("comp", j, "lo"))
        add(hi_u, c0, c1, ("comp", j, "hi"))
    for j, (lo_u, hi_u, _ci, comp_b, reg_b) in enumerate(_TERMS):
        c0, c1 = reg_b // 128, (reg_b + REG_LEN - 1) // 128
        add(lo_u, c0, c1, ("reg", j, "lo"))
        add(hi_u, c0, c1, ("reg", j, "hi"))
    bases, acc = [], 0
    for (_u, _c0, n) in segs:
        bases.append(acc)
        acc += n
    return segs, seg_of, bases, acc


_SEGS, _SEG_OF, _SEG_BASE, N_GATHER = _build_segments()


ZERO_ROW = T_STEPS * BT


def _prefix_body(x_ref, p_ref, carry_ref):
    t = pl.program_id(0)

    @pl.when(t == 0)
    def _():
        carry_ref[...] = jnp.zeros_like(carry_ref)

    @pl.when(t < T_STEPS)
    def _():
        xb_t = jnp.swapaxes(x_ref[...], 0, 1)
        row = lax.broadcasted_iota(jnp.int32, (BT, BT), 0)
        col = lax.broadcasted_iota(jnp.int32, (BT, BT), 1)
        tri = (col <= row).astype(jnp.float32)
        res = jnp.dot(tri, xb_t,
                      preferred_element_type=jnp.float32) + carry_ref[...]
        p_ref[...] = jnp.swapaxes(res.reshape(BT, N_CHUNKS, 128), 0, 1)
        carry_ref[...] = res[BT - 1:BT, :]

    @pl.when(t == T_STEPS)
    def _():
        p_ref[...] = jnp.zeros_like(p_ref)


_prefix_call = pl.pallas_call(
    _prefix_body,
    grid=(T_STEPS + 1,),
    in_specs=[pl.BlockSpec(
        (F_PAD, BT), lambda t: (0, jnp.minimum(t, T_STEPS - 1)))],
    out_specs=pl.BlockSpec((N_CHUNKS, BT, 128), lambda t: (0, t, 0)),
    out_shape=jax.ShapeDtypeStruct((N_CHUNKS, P_ROWS, 128), jnp.float32),
    scratch_shapes=[pltpu.VMEM((1, F_PAD), jnp.float32)],
    compiler_params=pltpu.CompilerParams(
        dimension_semantics=("arbitrary",)),
)


def _load_win(rows_v, seg_idx, rel_off):
    s = _SEG_BASE[seg_idx] * 128 + rel_off
    r0, c0 = divmod(s, 128)
    if c0 + L <= 128:
        return rows_v[r0, pl.ds(c0, L)]
    k = 128 - c0
    v0 = rows_v[r0, pl.ds(128 - L, L)]
    v1 = rows_v[r0 + 1, pl.ds(0, L)]
    lanes = lax.iota(jnp.int32, L)
    i0 = jnp.minimum(lanes + (c0 - (128 - L)), L - 1)
    i1 = jnp.maximum(lanes - k, 0)
    return jnp.where(lanes < k, _take16(v0, i0), _take16(v1, i1))


def _take16(v, idx):
    return lax.gather(
        v, idx[:, None],
        lax.GatherDimensionNumbers(
            offset_dims=(), collapsed_slice_dims=(0,), start_index_map=(0,)),
        slice_sizes=(1,),
        mode=lax.GatherScatterMode.PROMISE_IN_BOUNDS)


def _combine_body(p_hbm, idx_hbm, coefb_hbm, act_hbm, comp_hbm, reg_hbm,
                  idx_v, coefb_v, rows_v0, rows_v1,
                  act_v, comp_v, reg_v, sem0, sem1):
    wid = lax.axis_index("s") * NC + lax.axis_index("c")
    pltpu.sync_copy(coefb_hbm.at[wid], coefb_v)

    row_bufs = (rows_v0, rows_v1)
    sems = (sem0, sem1)
    pltpu.sync_copy(idx_hbm.at[wid], idx_v)
    copies = [pltpu.async_copy(p_hbm.at[idx_v.at[0]], rows_v0, sem0)]
    for slot in range(PROPS_PER_W):
        if slot + 1 < PROPS_PER_W:
            nb = (slot + 1) % 2
            copies.append(pltpu.async_copy(
                p_hbm.at[idx_v.at[slot + 1]], row_bufs[nb], sems[nb]))
        copies[slot].wait()
        rows_v = row_bufs[slot % 2]
        cbase = slot * N_COEF
        cf6 = [coefb_v[cbase + k, :] for k in range(N_COEF)]
        s_lo, s_hi = _SEG_OF[("act", 0, "lo")], _SEG_OF[("act", 0, "hi")]
        for c in range(ACT_PAD // L):
            off = c * L
            hi = _load_win(rows_v, s_hi, off)
            lo = _load_win(rows_v, s_lo, off)
            act_v[slot, pl.ds(off, L)] = (hi - lo) * cf6[0]
        for c in range(COMP_PAD // L):
            off = c * L
            acc = None
            for j, (_lo_u, _hi_u, ci, comp_b, _reg_b) in enumerate(_TERMS):
                rel = comp_b - 128 * _SEGS[_SEG_OF[("comp", j, "lo")]][1] + off
                hi = _load_win(rows_v, _SEG_OF[("comp", j, "hi")], rel)
                lo = _load_win(rows_v, _SEG_OF[("comp", j, "lo")], rel)
                term = (hi - lo) * cf6[ci]
                acc = term if acc is None else acc + term
            comp_v[slot, pl.ds(off, L)] = acc
        for c in range(REG_PAD // L):
            off = c * L
            acc = None
            for j, (_lo_u, _hi_u, ci, _comp_b, reg_b) in enumerate(_TERMS):
                rel = reg_b - 128 * _SEGS[_SEG_OF[("reg", j, "lo")]][1] + off
                hi = _load_win(rows_v, _SEG_OF[("reg", j, "hi")], rel)
                lo = _load_win(rows_v, _SEG_OF[("reg", j, "lo")], rel)
                term = (hi - lo) * cf6[ci]
                acc = term if acc is None else acc + term
            reg_v[slot, pl.ds(off, L)] = acc

    base = wid * PROPS_PER_W
    pltpu.sync_copy(act_v, act_hbm.at[pl.ds(base, PROPS_PER_W)])
    pltpu.sync_copy(comp_v, comp_hbm.at[pl.ds(base, PROPS_PER_W)])
    pltpu.sync_copy(reg_v, reg_hbm.at[pl.ds(base, PROPS_PER_W)])


@functools.cache
def _combine_call():
    return functools.partial(
        pl.kernel,
        mesh=plsc.VectorSubcoreMesh(core_axis_name="c", subcore_axis_name="s"),
        out_type=(
            jax.ShapeDtypeStruct((NUM_TICKS, ACT_PAD), jnp.float32),
            jax.ShapeDtypeStruct((NUM_TICKS, COMP_PAD), jnp.float32),
            jax.ShapeDtypeStruct((NUM_TICKS, REG_PAD), jnp.float32),
        ),
        scratch_types=[
            pltpu.VMEM((PROPS_PER_W, N_GATHER), jnp.int32),
            pltpu.VMEM((PROPS_PER_W * N_COEF, L), jnp.float32),
            pltpu.VMEM((N_GATHER, 128), jnp.float32),
            pltpu.VMEM((N_GATHER, 128), jnp.float32),
            pltpu.VMEM((PROPS_PER_W, ACT_PAD), jnp.float32),
            pltpu.VMEM((PROPS_PER_W, COMP_PAD), jnp.float32),
            pltpu.VMEM((PROPS_PER_W, REG_PAD), jnp.float32),
            pltpu.SemaphoreType.DMA,
            pltpu.SemaphoreType.DMA,
        ],
    )(_combine_body)


def _boundaries(proposal_ticks, scale_factors):
    tk = proposal_ticks.astype(jnp.int32)
    t0, t1, t2, t3 = tk[:, 0], tk[:, 1], tk[:, 2], tk[:, 3]
    r0 = jnp.maximum(t0 + 1, t1)
    r1 = jnp.maximum(t1 + 1, t2)
    r2 = jnp.maximum(t2 + 1, t3)
    m1 = t1 + (r1 - t1) // 2
    rows = jnp.stack([t0, r0, t1, m1, r1, t2, r2], axis=1)

    f32 = jnp.float32
    inv = lambda a, b: 1.0 / jnp.maximum(b - a, 1).astype(f32)
    coefs = jnp.stack([
        inv(t1, r1),
        scale_factors[:, 0] * inv(t0, r0),
        inv(t1, r1),
        inv(t1, m1),
        inv(m1, r1),
        scale_factors[:, 1] * inv(t2, r2),
    ], axis=1)
    return rows, coefs


_GATHER_U = np.concatenate(
    [np.full(n, u, np.int32) for (u, c0, n) in _SEGS])
_GATHER_C = np.concatenate(
    [np.arange(c0, c0 + n, dtype=np.int32) for (u, c0, n) in _SEGS])


def kernel(x, proposal_ticks, scale_factors):
    p3 = _prefix_call(jnp.swapaxes(x, 0, 1))
    p_flat = p3.reshape(N_CHUNKS * P_ROWS, 128)

    rows, coefs = _boundaries(proposal_ticks, scale_factors)
    rows = jnp.where(rows >= 1, rows - 1, ZERO_ROW)
    gat = rows[:, _GATHER_U] + jnp.asarray(_GATHER_C * P_ROWS)[None, :]
    idx = gat.reshape(NW, PROPS_PER_W, N_GATHER)
    coefb = jnp.broadcast_to(
        coefs[:, :, None], (NUM_TICKS, N_COEF, L)
    ).reshape(NW, PROPS_PER_W * N_COEF, L)

    act, comp, reg = _combine_call()(p_flat, idx, coefb)
    return act[:, :ACT_LEN], comp[:, :COMP_LEN], reg[:, :REG_LEN]


# speedup vs baseline: 3.0319x; 1.0121x over previous
import functools

import numpy as np
import jax
import jax.numpy as jnp
from jax import lax
from jax.experimental import pallas as pl
from jax.experimental.pallas import tpu as pltpu
from jax.experimental.pallas import tpu_sc as plsc

NUM_CLASSES = 200
ACT_LEN = NUM_CLASSES + 1
COMP_LEN = NUM_CLASSES
REG_LEN = NUM_CLASSES * 2
NUM_MULT = 5
FEAT_DIM = ACT_LEN + NUM_MULT * (COMP_LEN + REG_LEN)
T_TOTAL = 8192
NUM_TICKS = 128

F_PAD = 3328
N_CHUNKS = F_PAD // 128
BT = 512
T_STEPS = T_TOTAL // BT
P_ROWS = (T_STEPS + 1) * BT

NC, NS, L = 2, 16, 16
NW = NC * NS
PROPS_PER_W = NUM_TICKS // NW

ACT_PAD, COMP_PAD, REG_PAD = 208, 208, 416

U_L0, U_R0, U_L1, U_M1, U_R1, U_L2, U_R2 = range(7)

_TERMS = (
    (U_L0, U_R0, 1, 201, 1201),
    (U_L1, U_R1, 2, 401, 1601),
    (U_L1, U_M1, 3, 601, 2001),
    (U_M1, U_R1, 4, 801, 2401),
    (U_L2, U_R2, 5, 1001, 2801),
)
N_COEF = 6


def _build_segments():
    segs = []
    seg_of = {}
    def add(u, c0, c1, key):
        seg_of[key] = len(segs)
        segs.append((u, c0, c1 - c0 + 1))
    add(U_L1, 0, (ACT_LEN - 1) // 128, ("act", 0, "lo"))
    add(U_R1, 0, (ACT_LEN - 1) // 128, ("act", 0, "hi"))
    for j, (lo_u, hi_u, _ci, comp_b, reg_b) in enumerate(_TERMS):
        c0, c1 = comp_b // 128, (comp_b + COMP_LEN - 1) // 128
        add(lo_u, c0, c1, ("comp", j, "lo"))
        add(hi_u, c0, c1, ("comp", j, "hi"))
    for j, (lo_u, hi_u, _ci, comp_b, reg_b) in enumerate(_TERMS):
        c0, c1 = reg_b // 128, (reg_b + REG_LEN - 1) // 128
        add(lo_u, c0, c1, ("reg", j, "lo"))
        add(hi_u, c0, c1, ("reg", j, "hi"))
    bases, acc = [], 0
    for (_u, _c0, n) in segs:
        bases.append(acc)
        acc += n
    return segs, seg_of, bases, acc


_SEGS, _SEG_OF, _SEG_BASE, N_GATHER = _build_segments()


ZERO_ROW = T_STEPS * BT


def _prefix_body(x_ref, p_ref, carry_ref):
    t = pl.program_id(0)

    @pl.when(t == 0)
    def _():
        carry_ref[...] = jnp.zeros_like(carry_ref)

    @pl.when(t < T_STEPS)
    def _():
        xb_t = jnp.swapaxes(x_ref[...], 0, 1)
        row = lax.broadcasted_iota(jnp.int32, (BT, BT), 0)
        col = lax.broadcasted_iota(jnp.int32, (BT, BT), 1)
        tri = (col <= row).astype(jnp.float32)
        res = jnp.dot(tri, xb_t,
                      preferred_element_type=jnp.float32) + carry_ref[...]
        p_ref[...] = jnp.swapaxes(res.reshape(BT, N_CHUNKS, 128), 0, 1)
        carry_ref[...] = res[BT - 1:BT, :]

    @pl.when(t == T_STEPS)
    def _():
        p_ref[...] = jnp.zeros_like(p_ref)


_prefix_call = pl.pallas_call(
    _prefix_body,
    grid=(T_STEPS + 1,),
    in_specs=[pl.BlockSpec(
        (F_PAD, BT), lambda t: (0, jnp.minimum(t, T_STEPS - 1)))],
    out_specs=pl.BlockSpec((N_CHUNKS, BT, 128), lambda t: (0, t, 0)),
    out_shape=jax.ShapeDtypeStruct((N_CHUNKS, P_ROWS, 128), jnp.float32),
    scratch_shapes=[pltpu.VMEM((1, F_PAD), jnp.float32)],
    compiler_params=pltpu.CompilerParams(
        dimension_semantics=("arbitrary",)),
)


def _load_win(rows_v, seg_idx, rel_off):
    s = _SEG_BASE[seg_idx] * 128 + rel_off
    r0, c0 = divmod(s, 128)
    if c0 + L <= 128:
        return rows_v[r0, pl.ds(c0, L)]
    k = 128 - c0
    v0 = rows_v[r0, pl.ds(128 - L, L)]
    v1 = rows_v[r0 + 1, pl.ds(0, L)]
    lanes = lax.iota(jnp.int32, L)
    i0 = jnp.minimum(lanes + (c0 - (128 - L)), L - 1)
    i1 = jnp.maximum(lanes - k, 0)
    return jnp.where(lanes < k, _take16(v0, i0), _take16(v1, i1))


def _take16(v, idx):
    return lax.gather(
        v, idx[:, None],
        lax.GatherDimensionNumbers(
            offset_dims=(), collapsed_slice_dims=(0,), start_index_map=(0,)),
        slice_sizes=(1,),
        mode=lax.GatherScatterMode.PROMISE_IN_BOUNDS)


def _combine_body(p_hbm, idx_hbm, coefb_hbm, act_hbm, comp_hbm, reg_hbm,
                  idx_v, coefb_v, rows_v0, rows_v1,
                  act_v, comp_v, reg_v, sem0, sem1):
    wid = lax.axis_index("s") * NC + lax.axis_index("c")
    pltpu.sync_copy(coefb_hbm.at[wid], coefb_v)

    row_bufs = (rows_v0, rows_v1)
    sems = (sem0, sem1)
    pltpu.sync_copy(idx_hbm.at[wid], idx_v)
    copies = [pltpu.async_copy(p_hbm.at[idx_v.at[0]], rows_v0, sem0)]
    for slot in range(PROPS_PER_W):
        if slot + 1 < PROPS_PER_W:
            nb = (slot + 1) % 2
            copies.append(pltpu.async_copy(
                p_hbm.at[idx_v.at[slot + 1]], row_bufs[nb], sems[nb]))
        copies[slot].wait()
        rows_v = row_bufs[slot % 2]
        cbase = slot * N_COEF
        cf6 = [coefb_v[cbase + k, :] for k in range(N_COEF)]
        s_lo, s_hi = _SEG_OF[("act", 0, "lo")], _SEG_OF[("act", 0, "hi")]
        for c in range(ACT_PAD // L):
            off = c * L
            hi = _load_win(rows_v, s_hi, off)
            lo = _load_win(rows_v, s_lo, off)
            act_v[slot, pl.ds(off, L)] = (hi - lo) * cf6[0]
        for c in range(COMP_PAD // L):
            off = c * L
            acc = None
            for j, (_lo_u, _hi_u, ci, comp_b, _reg_b) in enumerate(_TERMS):
                rel = comp_b - 128 * _SEGS[_SEG_OF[("comp", j, "lo")]][1] + off
                hi = _load_win(rows_v, _SEG_OF[("comp", j, "hi")], rel)
                lo = _load_win(rows_v, _SEG_OF[("comp", j, "lo")], rel)
                term = (hi - lo) * cf6[ci]
                acc = term if acc is None else acc + term
            comp_v[slot, pl.ds(off, L)] = acc
        for c in range(REG_PAD // L):
            off = c * L
            acc = None
            for j, (_lo_u, _hi_u, ci, _comp_b, reg_b) in enumerate(_TERMS):
                rel = reg_b - 128 * _SEGS[_SEG_OF[("reg", j, "lo")]][1] + off
                hi = _load_win(rows_v, _SEG_OF[("reg", j, "hi")], rel)
                lo = _load_win(rows_v, _SEG_OF[("reg", j, "lo")], rel)
                term = (hi - lo) * cf6[ci]
                acc = term if acc is None else acc + term
            reg_v[slot, pl.ds(off, L)] = acc

    base = wid * PROPS_PER_W
    pltpu.sync_copy(act_v, act_hbm.at[pl.ds(base, PROPS_PER_W)])
    pltpu.sync_copy(comp_v, comp_hbm.at[pl.ds(base, PROPS_PER_W)])
    pltpu.sync_copy(reg_v, reg_hbm.at[pl.ds(base, PROPS_PER_W)])


@functools.cache
def _combine_call():
    return functools.partial(
        pl.kernel,
        mesh=plsc.VectorSubcoreMesh(core_axis_name="c", subcore_axis_name="s"),
        out_type=(
            jax.ShapeDtypeStruct((NUM_TICKS, ACT_PAD), jnp.float32),
            jax.ShapeDtypeStruct((NUM_TICKS, COMP_PAD), jnp.float32),
            jax.ShapeDtypeStruct((NUM_TICKS, REG_PAD), jnp.float32),
        ),
        scratch_types=[
            pltpu.VMEM((PROPS_PER_W, N_GATHER), jnp.int32),
            pltpu.VMEM((PROPS_PER_W * N_COEF, L), jnp.float32),
            pltpu.VMEM((N_GATHER, 128), jnp.float32),
            pltpu.VMEM((N_GATHER, 128), jnp.float32),
            pltpu.VMEM((PROPS_PER_W, ACT_PAD), jnp.float32),
            pltpu.VMEM((PROPS_PER_W, COMP_PAD), jnp.float32),
            pltpu.VMEM((PROPS_PER_W, REG_PAD), jnp.float32),
            pltpu.SemaphoreType.DMA,
            pltpu.SemaphoreType.DMA,
        ],
    )(_combine_body)


def _boundaries(proposal_ticks, scale_factors):
    tk = proposal_ticks.astype(jnp.int32)
    t0, t1, t2, t3 = tk[:, 0], tk[:, 1], tk[:, 2], tk[:, 3]
    r0 = jnp.maximum(t0 + 1, t1)
    r1 = jnp.maximum(t1 + 1, t2)
    r2 = jnp.maximum(t2 + 1, t3)
    m1 = t1 + (r1 - t1) // 2
    rows = jnp.stack([t0, r0, t1, m1, r1, t2, r2], axis=1)

    f32 = jnp.float32
    inv = lambda a, b: 1.0 / jnp.maximum(b - a, 1).astype(f32)
    coefs = jnp.stack([
        inv(t1, r1),
        scale_factors[:, 0] * inv(t0, r0),
        inv(t1, r1),
        inv(t1, m1),
        inv(m1, r1),
        scale_factors[:, 1] * inv(t2, r2),
    ], axis=1)
    return rows, coefs


_GATHER_U = np.concatenate(
    [np.full(n, u, np.int32) for (u, c0, n) in _SEGS])
_GATHER_C = np.concatenate(
    [np.arange(c0, c0 + n, dtype=np.int32) for (u, c0, n) in _SEGS])


def kernel(x, proposal_ticks, scale_factors):
    p3 = _prefix_call(jnp.swapaxes(x, 0, 1))
    p_flat = p3.reshape(N_CHUNKS * P_ROWS, 128)

    rows, coefs = _boundaries(proposal_ticks, scale_factors)
    rows = jnp.where(rows >= 1, rows - 1, ZERO_ROW)
    gat = rows[:, _GATHER_U] + jnp.asarray(_GATHER_C * P_ROWS)[None, :]
    idx = gat.reshape(NW, PROPS_PER_W, N_GATHER)
    coefb = jnp.broadcast_to(
        coefs[:, :, None], (NUM_TICKS, N_COEF, L)
    ).reshape(NW, PROPS_PER_W * N_COEF, L)

    act, comp, reg = _combine_call()(p_flat, idx, coefb)
    return act[:, :ACT_LEN], comp[:, :COMP_LEN], reg[:, :REG_LEN]


# speedup vs baseline: 3.0889x; 1.0188x over previous
import functools

import numpy as np
import jax
import jax.numpy as jnp
from jax import lax
from jax.experimental import pallas as pl
from jax.experimental.pallas import tpu as pltpu
from jax.experimental.pallas import tpu_sc as plsc

NUM_CLASSES = 200
ACT_LEN = NUM_CLASSES + 1
COMP_LEN = NUM_CLASSES
REG_LEN = NUM_CLASSES * 2
NUM_MULT = 5
FEAT_DIM = ACT_LEN + NUM_MULT * (COMP_LEN + REG_LEN)
T_TOTAL = 8192
NUM_TICKS = 128

F_PAD = 3328
N_CHUNKS = F_PAD // 128
BT = 512
T_STEPS = T_TOTAL // BT
P_ROWS = T_STEPS * BT

NC, NS, L = 2, 16, 16
NW = NC * NS
PROPS_PER_W = NUM_TICKS // NW

ACT_PAD, COMP_PAD, REG_PAD = 208, 208, 416

U_L0, U_R0, U_L1, U_M1, U_R1, U_L2, U_R2 = range(7)

_TERMS = (
    (U_L0, U_R0, 1, 201, 1201),
    (U_L1, U_R1, 2, 401, 1601),
    (U_L1, U_M1, 3, 601, 2001),
    (U_M1, U_R1, 4, 801, 2401),
    (U_L2, U_R2, 5, 1001, 2801),
)
N_COEF = 12


def _build_segments():
    segs = []
    seg_of = {}
    def add(u, c0, c1, key):
        seg_of[key] = len(segs)
        segs.append((u, c0, c1 - c0 + 1))
    add(U_L1, 0, (ACT_LEN - 1) // 128, ("act", 0, "lo"))
    add(U_R1, 0, (ACT_LEN - 1) // 128, ("act", 0, "hi"))
    for j, (lo_u, hi_u, _ci, comp_b, reg_b) in enumerate(_TERMS):
        c0, c1 = comp_b // 128, (comp_b + COMP_LEN - 1) // 128
        add(lo_u, c0, c1, ("comp", j, "lo"))
        add(hi_u, c0, c1, ("comp", j, "hi"))
    for j, (lo_u, hi_u, _ci, comp_b, reg_b) in enumerate(_TERMS):
        c0, c1 = reg_b // 128, (reg_b + REG_LEN - 1) // 128
        add(lo_u, c0, c1, ("reg", j, "lo"))
        add(hi_u, c0, c1, ("reg", j, "hi"))
    bases, acc = [], 0
    for (_u, _c0, n) in segs:
        bases.append(acc)
        acc += n
    return segs, seg_of, bases, acc


_SEGS, _SEG_OF, _SEG_BASE, N_GATHER = _build_segments()


def _prefix_body(x_ref, p_ref, carry_ref):
    t = pl.program_id(0)

    @pl.when(t == 0)
    def _():
        carry_ref[...] = jnp.zeros_like(carry_ref)

    xb_t = jnp.swapaxes(x_ref[...], 0, 1)
    row = lax.broadcasted_iota(jnp.int32, (BT, BT), 0)
    col = lax.broadcasted_iota(jnp.int32, (BT, BT), 1)
    tri = (col <= row).astype(jnp.float32)
    res = jnp.dot(tri, xb_t,
                  preferred_element_type=jnp.float32) + carry_ref[...]
    p_ref[...] = jnp.swapaxes(res.reshape(BT, N_CHUNKS, 128), 0, 1)
    carry_ref[...] = res[BT - 1:BT, :]


_prefix_call = pl.pallas_call(
    _prefix_body,
    grid=(T_STEPS,),
    in_specs=[pl.BlockSpec((F_PAD, BT), lambda t: (0, t))],
    out_specs=pl.BlockSpec((N_CHUNKS, BT, 128), lambda t: (0, t, 0)),
    out_shape=jax.ShapeDtypeStruct((N_CHUNKS, P_ROWS, 128), jnp.float32),
    scratch_shapes=[pltpu.VMEM((1, F_PAD), jnp.float32)],
    compiler_params=pltpu.CompilerParams(
        dimension_semantics=("arbitrary",)),
)


def _load_win(rows_v, seg_idx, rel_off):
    s = _SEG_BASE[seg_idx] * 128 + rel_off
    r0, c0 = divmod(s, 128)
    if c0 + L <= 128:
        return rows_v[r0, pl.ds(c0, L)]
    k = 128 - c0
    v0 = rows_v[r0, pl.ds(128 - L, L)]
    v1 = rows_v[r0 + 1, pl.ds(0, L)]
    lanes = lax.iota(jnp.int32, L)
    i0 = jnp.minimum(lanes + (c0 - (128 - L)), L - 1)
    i1 = jnp.maximum(lanes - k, 0)
    return jnp.where(lanes < k, _take16(v0, i0), _take16(v1, i1))


def _take16(v, idx):
    return lax.gather(
        v, idx[:, None],
        lax.GatherDimensionNumbers(
            offset_dims=(), collapsed_slice_dims=(0,), start_index_map=(0,)),
        slice_sizes=(1,),
        mode=lax.GatherScatterMode.PROMISE_IN_BOUNDS)


def _combine_body(p_hbm, idx_hbm, coefb_hbm, act_hbm, comp_hbm, reg_hbm,
                  idx_v, coefb_v, rows_v0, rows_v1,
                  act_v, comp_v, reg_v, sem0, sem1):
    wid = lax.axis_index("s") * NC + lax.axis_index("c")
    pltpu.sync_copy(coefb_hbm.at[wid], coefb_v)

    row_bufs = (rows_v0, rows_v1)
    sems = (sem0, sem1)
    pltpu.sync_copy(idx_hbm.at[wid], idx_v)
    copies = [pltpu.async_copy(p_hbm.at[idx_v.at[0]], rows_v0, sem0)]
    for slot in range(PROPS_PER_W):
        if slot + 1 < PROPS_PER_W:
            nb = (slot + 1) % 2
            copies.append(pltpu.async_copy(
                p_hbm.at[idx_v.at[slot + 1]], row_bufs[nb], sems[nb]))
        copies[slot].wait()
        rows_v = row_bufs[slot % 2]
        cbase = slot * N_COEF
        cf = [coefb_v[cbase + k, :] for k in range(N_COEF)]
        s_lo, s_hi = _SEG_OF[("act", 0, "lo")], _SEG_OF[("act", 0, "hi")]
        for c in range(ACT_PAD // L):
            off = c * L
            hi = _load_win(rows_v, s_hi, off)
            lo = _load_win(rows_v, s_lo, off)
            act_v[slot, pl.ds(off, L)] = hi * cf[0] - lo * cf[1]
        for c in range(COMP_PAD // L):
            off = c * L
            acc = None
            for j, (_lo_u, _hi_u, ci, comp_b, _reg_b) in enumerate(_TERMS):
                rel = comp_b - 128 * _SEGS[_SEG_OF[("comp", j, "lo")]][1] + off
                hi = _load_win(rows_v, _SEG_OF[("comp", j, "hi")], rel)
                lo = _load_win(rows_v, _SEG_OF[("comp", j, "lo")], rel)
                term = hi * cf[2 * ci] - lo * cf[2 * ci + 1]
                acc = term if acc is None else acc + term
            comp_v[slot, pl.ds(off, L)] = acc
        for c in range(REG_PAD // L):
            off = c * L
            acc = None
            for j, (_lo_u, _hi_u, ci, _comp_b, reg_b) in enumerate(_TERMS):
                rel = reg_b - 128 * _SEGS[_SEG_OF[("reg", j, "lo")]][1] + off
                hi = _load_win(rows_v, _SEG_OF[("reg", j, "hi")], rel)
                lo = _load_win(rows_v, _SEG_OF[("reg", j, "lo")], rel)
                term = hi * cf[2 * ci] - lo * cf[2 * ci + 1]
                acc = term if acc is None else acc + term
            reg_v[slot, pl.ds(off, L)] = acc

    base = wid * PROPS_PER_W
    pltpu.sync_copy(act_v, act_hbm.at[pl.ds(base, PROPS_PER_W)])
    pltpu.sync_copy(comp_v, comp_hbm.at[pl.ds(base, PROPS_PER_W)])
    pltpu.sync_copy(reg_v, reg_hbm.at[pl.ds(base, PROPS_PER_W)])


@functools.cache
def _combine_call():
    return functools.partial(
        pl.kernel,
        mesh=plsc.VectorSubcoreMesh(core_axis_name="c", subcore_axis_name="s"),
        out_type=(
            jax.ShapeDtypeStruct((NUM_TICKS, ACT_PAD), jnp.float32),
            jax.ShapeDtypeStruct((NUM_TICKS, COMP_PAD), jnp.float32),
            jax.ShapeDtypeStruct((NUM_TICKS, REG_PAD), jnp.float32),
        ),
        scratch_types=[
            pltpu.VMEM((PROPS_PER_W, N_GATHER), jnp.int32),
            pltpu.VMEM((PROPS_PER_W * N_COEF, L), jnp.float32),
            pltpu.VMEM((N_GATHER, 128), jnp.float32),
            pltpu.VMEM((N_GATHER, 128), jnp.float32),
            pltpu.VMEM((PROPS_PER_W, ACT_PAD), jnp.float32),
            pltpu.VMEM((PROPS_PER_W, COMP_PAD), jnp.float32),
            pltpu.VMEM((PROPS_PER_W, REG_PAD), jnp.float32),
            pltpu.SemaphoreType.DMA,
            pltpu.SemaphoreType.DMA,
        ],
    )(_combine_body)


def _boundaries(proposal_ticks, scale_factors):
    tk = proposal_ticks.astype(jnp.int32)
    t0, t1, t2, t3 = tk[:, 0], tk[:, 1], tk[:, 2], tk[:, 3]
    r0 = jnp.maximum(t0 + 1, t1)
    r1 = jnp.maximum(t1 + 1, t2)
    r2 = jnp.maximum(t2 + 1, t3)
    m1 = t1 + (r1 - t1) // 2
    rows = jnp.stack([t0, r0, t1, m1, r1, t2, r2], axis=1)

    f32 = jnp.float32
    inv = lambda a, b: 1.0 / jnp.maximum(b - a, 1).astype(f32)
    cf_hi = [
        inv(t1, r1),
        scale_factors[:, 0] * inv(t0, r0),
        inv(t1, r1),
        inv(t1, m1),
        inv(m1, r1),
        scale_factors[:, 1] * inv(t2, r2),
    ]
    hi_rows = [r1, r0, r1, m1, r1, r2]
    lo_rows = [t1, t0, t1, t1, m1, t2]
    pairs = []
    for k in range(6):
        pairs.append(cf_hi[k] * (hi_rows[k] != 0).astype(f32))
        pairs.append(cf_hi[k] * (lo_rows[k] != 0).astype(f32))
    coefs = jnp.stack(pairs, axis=1)
    return rows, coefs


_GATHER_U = np.concatenate(
    [np.full(n, u, np.int32) for (u, c0, n) in _SEGS])
_GATHER_C = np.concatenate(
    [np.arange(c0, c0 + n, dtype=np.int32) for (u, c0, n) in _SEGS])


def kernel(x, proposal_ticks, scale_factors):
    p3 = _prefix_call(jnp.swapaxes(x, 0, 1))
    p_flat = p3.reshape(N_CHUNKS * P_ROWS, 128)

    rows, coefs = _boundaries(proposal_ticks, scale_factors)
    rows = jnp.maximum(rows - 1, 0)
    gat = rows[:, _GATHER_U] + jnp.asarray(_GATHER_C * P_ROWS)[None, :]
    idx = gat.reshape(NW, PROPS_PER_W, N_GATHER)
    coefb = jnp.broadcast_to(
        coefs[:, :, None], (NUM_TICKS, N_COEF, L)
    ).reshape(NW, PROPS_PER_W * N_COEF, L)

    act, comp, reg = _combine_call()(p_flat, idx, coefb)
    return act[:, :ACT_LEN], comp[:, :COMP_LEN], reg[:, :REG_LEN]
